# baseline (speedup 1.0000x reference)
import functools

import jax
import jax.numpy as jnp
import numpy as np
from jax.experimental import pallas as pl
from jax.experimental.pallas import tpu as pltpu

_C1 = 16
_K1, _S1 = 8, 4
_HEADW = 128
_NOUT = 6


def _cdiv(a, b):
    return (a + b - 1) // b


def _leaky(v):
    return jnp.maximum(v, 0.01 * v)


def _fused_kernel(x_ref, w01_ref, bias_ref, w2_ref, w3_ref, wh_ref,
                  oa_ref, ov_ref, *,
                  tb, c_in, h_in, w_in, oh1, n1, flat2, hidden, num_outputs):
    xv = x_ref[...].reshape(tb, c_in * h_in * w_in)
    jh_n = h_in // _S1
    lane_jh = _S1 * w_in
    lane_c = h_in * w_in
    kl = c_in * _S1 * w_in

    b1 = bias_ref[:, 0:n1]
    b2 = bias_ref[:, n1:n1 + flat2]
    b3 = bias_ref[:, n1 + flat2:n1 + flat2 + hidden]
    bh = bias_ref[:, n1 + flat2 + hidden:n1 + flat2 + hidden + _HEADW]

    slabs = []
    for jh in range(jh_n):
        parts = [xv[:, c * lane_c + jh * lane_jh:
                    c * lane_c + (jh + 1) * lane_jh].astype(jnp.bfloat16)
                 for c in range(c_in)]
        slabs.append(jnp.concatenate(parts, axis=1))
    s_all = jnp.concatenate(slabs, axis=0)

    h1 = (jnp.dot(s_all[0:oh1 * tb], w01_ref[0:kl],
                  preferred_element_type=jnp.float32)
          + jnp.dot(s_all[tb:(oh1 + 1) * tb], w01_ref[kl:2 * kl],
                    preferred_element_type=jnp.float32)
          + b1)
    h1 = _leaky(h1).astype(jnp.bfloat16)

    f = b2 + jnp.zeros((tb, flat2), jnp.float32)
    for oh in range(oh1):
        f = f + jnp.dot(h1[oh * tb:(oh + 1) * tb, :],
                        w2_ref[oh * n1:(oh + 1) * n1, :],
                        preferred_element_type=jnp.float32)
    f = _leaky(f).astype(jnp.bfloat16)

    hid = _leaky(jnp.dot(f, w3_ref[...], preferred_element_type=jnp.float32)
                 + b3).astype(jnp.bfloat16)
    head = jnp.dot(hid, wh_ref[...], preferred_element_type=jnp.float32) + bh

    col = jax.lax.broadcasted_iota(jnp.int32, head.shape, 1)
    amask = (col >= 1) & (col < 1 + num_outputs)
    logits = jnp.where(amask, head, jnp.float32(-1e30))
    m = jnp.max(logits, axis=1, keepdims=True)
    e = jnp.where(amask, jnp.exp(logits - m), 0.0)
    inv = pl.reciprocal(jnp.sum(e, axis=1, keepdims=True), approx=False)
    probs = e * inv
    oa_ref[...] = probs[:, 1:1 + num_outputs]
    ov_ref[...] = head[:, 0:1]


def _conv1_tap_weights(w1col, c, w, ow1):
    wi = np.arange(w)[:, None, None]
    ow = np.arange(ow1)[None, :, None]
    kw = np.arange(_K1)[None, None, :]
    e = jnp.asarray((wi == _S1 * ow + kw).astype(np.float32))
    w1r = w1col.reshape(c, 2, _S1, _K1, _C1)
    tap = jnp.einsum('cpuko,wak->pcuwao', w1r, e)
    return tap.reshape(2 * c * _S1 * w, ow1 * _C1).astype(jnp.bfloat16)


@jax.jit
def kernel(x, w1col, b1row, w2dense, b2row, w3k, b3row, whead, bhead, log_std):
    B, C, H, W = x.shape
    oh1, ow1 = (H - _K1) // _S1 + 1, (W - _K1) // _S1 + 1
    n1 = ow1 * _C1
    flat2 = w2dense.shape[1]
    hidden = w3k.shape[1]

    xflat = x
    tb = 128
    bp = _cdiv(B, tb) * tb
    if bp != B:
        xflat = jnp.pad(xflat, ((0, bp - B), (0, 0), (0, 0), (0, 0)))

    w01 = _conv1_tap_weights(w1col, C, W, ow1)
    bias = jnp.concatenate(
        [jnp.tile(b1row, (1, ow1)), b2row, b3row, bhead], axis=1)
    w2b = w2dense.astype(jnp.bfloat16)
    w3b = w3k.astype(jnp.bfloat16)
    whb = whead.astype(jnp.bfloat16)

    act, value = pl.pallas_call(
        functools.partial(_fused_kernel, tb=tb, c_in=C, h_in=H, w_in=W,
                          oh1=oh1, n1=n1, flat2=flat2, hidden=hidden,
                          num_outputs=_NOUT),
        out_shape=(jax.ShapeDtypeStruct((bp, _NOUT), jnp.float32),
                   jax.ShapeDtypeStruct((bp, 1), jnp.float32)),
        grid=(bp // tb,),
        in_specs=[
            pl.BlockSpec((tb, C, H, W), lambda i: (i, 0, 0, 0)),
            pl.BlockSpec((2 * C * _S1 * W, n1), lambda i: (0, 0)),
            pl.BlockSpec((1, n1 + flat2 + hidden + _HEADW), lambda i: (0, 0)),
            pl.BlockSpec((oh1 * n1, flat2), lambda i: (0, 0)),
            pl.BlockSpec((flat2, hidden), lambda i: (0, 0)),
            pl.BlockSpec((hidden, _HEADW), lambda i: (0, 0)),
        ],
        out_specs=(pl.BlockSpec((tb, _NOUT), lambda i: (i, 0)),
                   pl.BlockSpec((tb, 1), lambda i: (i, 0))),
        compiler_params=pltpu.CompilerParams(dimension_semantics=("parallel",)),
    )(xflat, w01, bias, w2b, w3b, whb)

    if bp != B:
        act, value = act[:B], value[:B]
    return act, value


# speedup vs baseline: 1.5219x; 1.5219x over previous
import functools

import jax
import jax.numpy as jnp
import numpy as np
from jax.experimental import pallas as pl
from jax.experimental.pallas import tpu as pltpu

_C1 = 16
_K1, _S1 = 8, 4
_HEADW = 128
_NOUT = 6


def _cdiv(a, b):
    return (a + b - 1) // b


def _leaky(v):
    return jnp.maximum(v, 0.01 * v)


def _fused_kernel(x_ref, w01_ref, bias_ref, w2_ref, w3_ref, wh_ref,
                  oa_ref, ov_ref, *,
                  tb, c_in, h_in, w_in, oh1, n1, flat2, hidden, num_outputs):
    jh_n = h_in // _S1
    lane_jh = _S1 * w_in
    lane_c = h_in * w_in
    kl = c_in * _S1 * w_in

    b1 = bias_ref[:, 0:n1]
    b2 = bias_ref[:, n1:n1 + flat2]
    b3 = bias_ref[:, n1 + flat2:n1 + flat2 + hidden]
    bh = bias_ref[:, n1 + flat2 + hidden:n1 + flat2 + hidden + _HEADW]

    slabs = []
    for jh in range(jh_n):
        parts = [x_ref[:, c * lane_c + jh * lane_jh:
                       c * lane_c + (jh + 1) * lane_jh].astype(jnp.bfloat16)
                 for c in range(c_in)]
        slabs.append(jnp.concatenate(parts, axis=1))
    s_all = jnp.concatenate(slabs, axis=0)

    h1 = (jnp.dot(s_all[0:oh1 * tb], w01_ref[0:kl],
                  preferred_element_type=jnp.float32)
          + jnp.dot(s_all[tb:(oh1 + 1) * tb], w01_ref[kl:2 * kl],
                    preferred_element_type=jnp.float32)
          + b1)
    h1 = _leaky(h1).astype(jnp.bfloat16)

    parts_f = [jnp.dot(h1[oh * tb:(oh + 1) * tb, :],
                       w2_ref[oh * n1:(oh + 1) * n1, :],
                       preferred_element_type=jnp.float32)
               for oh in range(oh1)]
    parts_f.append(b2 + jnp.zeros((tb, flat2), jnp.float32))
    while len(parts_f) > 1:
        parts_f = [a + b for a, b in zip(parts_f[::2], parts_f[1::2])] +             (parts_f[-1:] if len(parts_f) % 2 else [])
    f = _leaky(parts_f[0]).astype(jnp.bfloat16)

    hid = _leaky(jnp.dot(f, w3_ref[...], preferred_element_type=jnp.float32)
                 + b3).astype(jnp.bfloat16)
    head = jnp.dot(hid, wh_ref[...], preferred_element_type=jnp.float32) + bh

    col = jax.lax.broadcasted_iota(jnp.int32, head.shape, 1)
    amask = (col >= 1) & (col < 1 + num_outputs)
    logits = jnp.where(amask, head, jnp.float32(-1e30))
    m = jnp.max(logits, axis=1, keepdims=True)
    e = jnp.where(amask, jnp.exp(logits - m), 0.0)
    inv = pl.reciprocal(jnp.sum(e, axis=1, keepdims=True), approx=False)
    probs = e * inv
    oa_ref[...] = probs[:, 1:1 + num_outputs]
    ov_ref[...] = head[:, 0:1]


def _conv1_tap_weights(w1col, c, w, ow1):
    wi = np.arange(w)[:, None, None]
    ow = np.arange(ow1)[None, :, None]
    kw = np.arange(_K1)[None, None, :]
    e = jnp.asarray((wi == _S1 * ow + kw).astype(np.float32))
    w1r = w1col.reshape(c, 2, _S1, _K1, _C1)
    tap = jnp.einsum('cpuko,wak->pcuwao', w1r, e)
    return tap.reshape(2 * c * _S1 * w, ow1 * _C1).astype(jnp.bfloat16)


@jax.jit
def kernel(x, w1col, b1row, w2dense, b2row, w3k, b3row, whead, bhead, log_std):
    B, C, H, W = x.shape
    oh1, ow1 = (H - _K1) // _S1 + 1, (W - _K1) // _S1 + 1
    n1 = ow1 * _C1
    flat2 = w2dense.shape[1]
    hidden = w3k.shape[1]

    xflat = x.reshape(B, C * H * W)

    tb = 128
    bp = _cdiv(B, tb) * tb
    if bp != B:
        xflat = jnp.pad(xflat, ((0, bp - B), (0, 0)))

    w01 = _conv1_tap_weights(w1col, C, W, ow1)
    bias = jnp.concatenate(
        [jnp.tile(b1row, (1, ow1)), b2row, b3row, bhead], axis=1)
    w2b = w2dense.astype(jnp.bfloat16)
    w3b = w3k.astype(jnp.bfloat16)
    whb = whead.astype(jnp.bfloat16)

    act, value = pl.pallas_call(
        functools.partial(_fused_kernel, tb=tb, c_in=C, h_in=H, w_in=W,
                          oh1=oh1, n1=n1, flat2=flat2, hidden=hidden,
                          num_outputs=_NOUT),
        out_shape=(jax.ShapeDtypeStruct((bp, _NOUT), jnp.float32),
                   jax.ShapeDtypeStruct((bp, 1), jnp.float32)),
        grid=(bp // tb,),
        in_specs=[
            pl.BlockSpec((tb, C * H * W), lambda i: (i, 0)),
            pl.BlockSpec((2 * C * _S1 * W, n1), lambda i: (0, 0)),
            pl.BlockSpec((1, n1 + flat2 + hidden + _HEADW), lambda i: (0, 0)),
            pl.BlockSpec((oh1 * n1, flat2), lambda i: (0, 0)),
            pl.BlockSpec((flat2, hidden), lambda i: (0, 0)),
            pl.BlockSpec((hidden, _HEADW), lambda i: (0, 0)),
        ],
        out_specs=(pl.BlockSpec((tb, _NOUT), lambda i: (i, 0)),
                   pl.BlockSpec((tb, 1), lambda i: (i, 0))),
        compiler_params=pltpu.CompilerParams(dimension_semantics=("parallel",)),
    )(xflat, w01, bias, w2b, w3b, whb)

    if bp != B:
        act, value = act[:B], value[:B]
    return act, value
